# baseline (speedup 1.0000x reference)
import functools

import jax
import jax.numpy as jnp
from jax.experimental import pallas as pl
from jax.experimental.layout import Format, Layout, with_layout_constraint
from jax.experimental.pallas import tpu as pltpu
from jax.experimental.pallas import tpu_sc as plsc

_F = 26
_V = 100000
_D = 32
_B = 16384
_N = _B * _F
_W = 1024


def _sc_gather(flat_table, flat_idx):
  mesh = plsc.VectorSubcoreMesh(
      core_axis_name="core", subcore_axis_name="subcore")

  @functools.partial(
      pl.kernel,
      out_type=jax.ShapeDtypeStruct((_N, _D), jnp.float32),
      mesh=mesh,
      compiler_params=pltpu.CompilerParams(use_tc_tiling_on_sc=False),
  )
  def k(table_hbm, idx_hbm, out_hbm):
    def body(i_vmem, o_vmem):
      pltpu.sync_copy(table_hbm.at[i_vmem.at[0]], o_vmem)

    pltpu.emit_pipeline(
        body,
        grid=(_N // _W,),
        in_specs=[pl.BlockSpec((1, _W), index_map=lambda i: (0, i))],
        out_specs=[pl.BlockSpec((_W, _D), index_map=lambda i: (i, 0))],
        core_axis_name=("core", "subcore"),
        dimension_semantics=(pltpu.PARALLEL,),
    )(idx_hbm, out_hbm)

  return k(flat_table, flat_idx)


@jax.jit
def kernel(features, tables):
  t = jnp.swapaxes(tables, 1, 2)
  t = t.reshape(_F * _D, _V).T
  flat_table = t.reshape(_F * _V, _D)
  flat_idx = features.astype(jnp.int32) * _F + jnp.arange(
      _F, dtype=jnp.int32)[None, :]
  flat_idx = flat_idx.reshape(1, _N)
  out = _sc_gather(flat_table, flat_idx)
  return out.reshape(_B, _F * _D)


# speedup vs baseline: 1.5371x; 1.5371x over previous
import functools

import jax
import jax.numpy as jnp
from jax.experimental import pallas as pl
from jax.experimental.pallas import tpu as pltpu
from jax.experimental.pallas import tpu_sc as plsc

_F = 26
_V = 100000
_D = 32
_B = 16384
_N = _B * _F
_W = 1024
_RB = 4000


def _tc_repack(t_padded):
  def body(x_ref, o_ref):
    for q in range(4):
      o_ref[:, pl.dslice(q * _D, _D)] = x_ref[pl.Slice(q, _RB // 4, 4), :]

  return pl.pallas_call(
      body,
      grid=(_F * _V // _RB,),
      in_specs=[pl.BlockSpec((_RB, _D), lambda i: (i, 0))],
      out_specs=pl.BlockSpec((_RB // 4, 4 * _D), lambda i: (i, 0)),
      out_shape=jax.ShapeDtypeStruct((_F * _V // 4, 4 * _D), jnp.float32),
      compiler_params=pltpu.CompilerParams(
          dimension_semantics=("parallel",)),
  )(t_padded)


def _sc_gather(flat_table, flat_idx):
  mesh = plsc.VectorSubcoreMesh(
      core_axis_name="core", subcore_axis_name="subcore")

  @functools.partial(
      pl.kernel,
      out_type=jax.ShapeDtypeStruct((_N, _D), jnp.float32),
      mesh=mesh,
      compiler_params=pltpu.CompilerParams(use_tc_tiling_on_sc=False),
  )
  def k(table_hbm, idx_hbm, out_hbm):
    def body(i_vmem, o_vmem):
      pltpu.sync_copy(table_hbm.at[i_vmem.at[0]], o_vmem)

    pltpu.emit_pipeline(
        body,
        grid=(_N // _W,),
        in_specs=[pl.BlockSpec((1, _W), index_map=lambda i: (0, i))],
        out_specs=[pl.BlockSpec((_W, _D), index_map=lambda i: (i, 0))],
        core_axis_name=("core", "subcore"),
        dimension_semantics=(pltpu.PARALLEL,),
    )(idx_hbm, out_hbm)

  return k(flat_table, flat_idx)


@jax.jit
def kernel(features, tables):
  flat_padded = tables.reshape(_F * _V, _D)
  flat_table = _tc_repack(flat_padded).reshape(_F * _V, _D)
  flat_idx = features.astype(jnp.int32) + (
      jnp.arange(_F, dtype=jnp.int32) * _V)[None, :]
  flat_idx = flat_idx.reshape(1, _N)
  out = _sc_gather(flat_table, flat_idx)
  return out.reshape(_B, _F * _D)


# speedup vs baseline: 1.9669x; 1.2797x over previous
import functools

import jax
import jax.numpy as jnp
from jax.experimental import pallas as pl
from jax.experimental.pallas import tpu as pltpu
from jax.experimental.pallas import tpu_sc as plsc

_F = 26
_V = 100000
_D = 32
_B = 16384
_N = _B * _F
_W = 1024
_RB = 16000


def _tc_repack(t_padded):
  def body(x_ref, o_ref):
    for q in range(4):
      o_ref[:, pl.dslice(q * _D, _D)] = x_ref[pl.Slice(q, _RB // 4, 4), :]

  return pl.pallas_call(
      body,
      grid=(_F * _V // _RB,),
      in_specs=[pl.BlockSpec((_RB, _D), lambda i: (i, 0))],
      out_specs=pl.BlockSpec((_RB // 4, 4 * _D), lambda i: (i, 0)),
      out_shape=jax.ShapeDtypeStruct((_F * _V // 4, 4 * _D), jnp.float32),
      compiler_params=pltpu.CompilerParams(
          dimension_semantics=("parallel",)),
  )(t_padded)


def _sc_gather(flat_table, flat_idx):
  mesh = plsc.VectorSubcoreMesh(
      core_axis_name="core", subcore_axis_name="subcore")

  @functools.partial(
      pl.kernel,
      out_type=jax.ShapeDtypeStruct((_N, _D), jnp.float32),
      mesh=mesh,
      compiler_params=pltpu.CompilerParams(use_tc_tiling_on_sc=False),
  )
  def k(table_hbm, idx_hbm, out_hbm):
    def body(i_vmem, o_vmem):
      pltpu.sync_copy(table_hbm.at[i_vmem.at[0]], o_vmem)

    pltpu.emit_pipeline(
        body,
        grid=(_N // _W,),
        in_specs=[pl.BlockSpec((1, _W), index_map=lambda i: (0, i))],
        out_specs=[pl.BlockSpec((_W, _D), index_map=lambda i: (i, 0))],
        core_axis_name=("core", "subcore"),
        dimension_semantics=(pltpu.PARALLEL,),
    )(idx_hbm, out_hbm)

  return k(flat_table, flat_idx)


@jax.jit
def kernel(features, tables):
  flat_padded = tables.reshape(_F * _V, _D)
  flat_table = _tc_repack(flat_padded).reshape(_F * _V, _D)
  flat_idx = features.astype(jnp.int32) + (
      jnp.arange(_F, dtype=jnp.int32) * _V)[None, :]
  flat_idx = flat_idx.reshape(1, _N)
  out = _sc_gather(flat_table, flat_idx)
  return out.reshape(_B, _F * _D)


# speedup vs baseline: 1.9724x; 1.0028x over previous
import functools

import jax
import jax.numpy as jnp
from jax.experimental import pallas as pl
from jax.experimental.pallas import tpu as pltpu
from jax.experimental.pallas import tpu_sc as plsc

_F = 26
_V = 100000
_D = 32
_B = 16384
_N = _B * _F
_W = 1024
_RB = 20800


def _tc_repack(t_padded):
  def body(x_ref, o_ref):
    for q in range(4):
      o_ref[:, pl.dslice(q * _D, _D)] = x_ref[pl.Slice(q, _RB // 4, 4), :]

  return pl.pallas_call(
      body,
      grid=(_F * _V // _RB,),
      in_specs=[pl.BlockSpec((_RB, _D), lambda i: (i, 0))],
      out_specs=pl.BlockSpec((_RB // 4, 4 * _D), lambda i: (i, 0)),
      out_shape=jax.ShapeDtypeStruct((_F * _V // 4, 4 * _D), jnp.float32),
      compiler_params=pltpu.CompilerParams(
          dimension_semantics=("parallel",)),
  )(t_padded)


def _sc_gather(flat_table, flat_idx):
  mesh = plsc.VectorSubcoreMesh(
      core_axis_name="core", subcore_axis_name="subcore")

  @functools.partial(
      pl.kernel,
      out_type=jax.ShapeDtypeStruct((_N, _D), jnp.float32),
      mesh=mesh,
      compiler_params=pltpu.CompilerParams(use_tc_tiling_on_sc=False),
  )
  def k(table_hbm, idx_hbm, out_hbm):
    def body(i_vmem, o_vmem):
      pltpu.sync_copy(table_hbm.at[i_vmem.at[0]], o_vmem)

    pltpu.emit_pipeline(
        body,
        grid=(_N // _W,),
        in_specs=[pl.BlockSpec((1, _W), index_map=lambda i: (0, i))],
        out_specs=[pl.BlockSpec((_W, _D), index_map=lambda i: (i, 0))],
        core_axis_name=("core", "subcore"),
        dimension_semantics=(pltpu.PARALLEL,),
    )(idx_hbm, out_hbm)

  return k(flat_table, flat_idx)


@jax.jit
def kernel(features, tables):
  flat_padded = tables.reshape(_F * _V, _D)
  flat_table = _tc_repack(flat_padded).reshape(_F * _V, _D)
  flat_idx = features.astype(jnp.int32) + (
      jnp.arange(_F, dtype=jnp.int32) * _V)[None, :]
  flat_idx = flat_idx.reshape(1, _N)
  out = _sc_gather(flat_table, flat_idx)
  return out.reshape(_B, _F * _D)


# speedup vs baseline: 1.9788x; 1.0033x over previous
import functools

import jax
import jax.numpy as jnp
from jax.experimental import pallas as pl
from jax.experimental.pallas import tpu as pltpu
from jax.experimental.pallas import tpu_sc as plsc

_F = 26
_V = 100000
_D = 32
_B = 16384
_N = _B * _F
_W = 1024
_RB = 40000


def _tc_repack(t_padded):
  def body(x_ref, o_ref):
    for q in range(4):
      o_ref[:, pl.dslice(q * _D, _D)] = x_ref[pl.Slice(q, _RB // 4, 4), :]

  return pl.pallas_call(
      body,
      grid=(_F * _V // _RB,),
      in_specs=[pl.BlockSpec((_RB, _D), lambda i: (i, 0))],
      out_specs=pl.BlockSpec((_RB // 4, 4 * _D), lambda i: (i, 0)),
      out_shape=jax.ShapeDtypeStruct((_F * _V // 4, 4 * _D), jnp.float32),
      compiler_params=pltpu.CompilerParams(
          dimension_semantics=("parallel",)),
  )(t_padded)


def _sc_gather(flat_table, flat_idx):
  mesh = plsc.VectorSubcoreMesh(
      core_axis_name="core", subcore_axis_name="subcore")

  @functools.partial(
      pl.kernel,
      out_type=jax.ShapeDtypeStruct((_N, _D), jnp.float32),
      mesh=mesh,
      compiler_params=pltpu.CompilerParams(use_tc_tiling_on_sc=False),
  )
  def k(table_hbm, idx_hbm, out_hbm):
    def body(i_vmem, o_vmem):
      pltpu.sync_copy(table_hbm.at[i_vmem.at[0]], o_vmem)

    pltpu.emit_pipeline(
        body,
        grid=(_N // _W,),
        in_specs=[pl.BlockSpec((1, _W), index_map=lambda i: (0, i))],
        out_specs=[pl.BlockSpec((_W, _D), index_map=lambda i: (i, 0))],
        core_axis_name=("core", "subcore"),
        dimension_semantics=(pltpu.PARALLEL,),
    )(idx_hbm, out_hbm)

  return k(flat_table, flat_idx)


@jax.jit
def kernel(features, tables):
  flat_padded = tables.reshape(_F * _V, _D)
  flat_table = _tc_repack(flat_padded).reshape(_F * _V, _D)
  flat_idx = features.astype(jnp.int32) + (
      jnp.arange(_F, dtype=jnp.int32) * _V)[None, :]
  flat_idx = flat_idx.reshape(1, _N)
  out = _sc_gather(flat_table, flat_idx)
  return out.reshape(_B, _F * _D)
